# Initial kernel scaffold; baseline (speedup 1.0000x reference)
import jax
import jax.numpy as jnp
from jax.experimental import pallas as pl


def kernel(coords, i, j, k, l, force, period, phase):
    raise NotImplementedError("write your pallas kernel here")


# speedup vs baseline: 4.8037x; 4.8037x over previous
import functools

import jax
import jax.numpy as jnp
import numpy as np
from jax import lax
from jax.experimental import pallas as pl
from jax.experimental.pallas import tpu as pltpu
from jax.experimental.pallas import tpu_sc as plsc

N_ATOMS = 100000
N_DIH = 500000
NC = 2
NS = 16
L = 16
NW = NC * NS
PER_W = 16384
NPAD = NW * PER_W
BLK = 128
NBLK_CHUNK = 16
CHUNK = BLK * NBLK_CHUNK
NCHUNK = PER_W // CHUNK
GROUPS = CHUNK // L
NSTR = 12

_SIN = (1.0, -1.6666651e-01, 8.3329640e-03, -1.9804748e-04, 2.5980951e-06)
_COS = (1.0, -0.5, 4.1666642e-02, -1.3888433e-03, 2.4763767e-05, -2.6114949e-07)
_HALF_PI = np.float32(1.5707964)


def _rsqrt(q):
    xi = lax.bitcast_convert_type(q, jnp.int32)
    yi = jnp.int32(0x5F3759DF) - lax.shift_right_logical(xi, 1)
    y = lax.bitcast_convert_type(yi, jnp.float32)
    h = q * np.float32(0.5)
    y = y * (np.float32(1.5) - h * y * y)
    y = y * (np.float32(1.5) - h * y * y)
    return y


def _sc_body(cflat_ref, idx_ref, f_ref, per_ref, ph_ref,
             out_ref, idx_v, f_v, per_v, ph_v, gbuf, acc_v, sem):
    cidx = lax.axis_index("c")
    sidx = lax.axis_index("s")
    wid = sidx * NC + cidx
    blk0 = wid * (PER_W // BLK)
    el0 = wid * PER_W
    f1 = np.float32(1.0)

    def chunk_body(cc, acc):
        row0 = blk0 + cc * NBLK_CHUNK
        e0 = el0 + cc * CHUNK
        pltpu.sync_copy(idx_ref.at[pl.ds(row0, NBLK_CHUNK)], idx_v)
        pltpu.sync_copy(f_ref.at[pl.ds(e0, CHUNK)], f_v)
        pltpu.sync_copy(per_ref.at[pl.ds(e0, CHUNK)], per_v)
        pltpu.sync_copy(ph_ref.at[pl.ds(e0, CHUNK)], ph_v)

        @pl.loop(0, NBLK_CHUNK)
        def _gather(b):
            dsts = pl.ds(b * BLK, BLK)
            handles = []
            for c in range(NSTR):
                handles.append(pltpu.async_copy(
                    cflat_ref.at[idx_v.at[b, c]], gbuf.at[c, dsts], sem))
            for h in handles:
                h.wait()

        def group_body(g, acc):
            base = g * L
            sl = pl.ds(base, L)
            p0x = gbuf[0, sl]
            p0y = gbuf[1, sl]
            p0z = gbuf[2, sl]
            p1x = gbuf[3, sl]
            p1y = gbuf[4, sl]
            p1z = gbuf[5, sl]
            p2x = gbuf[6, sl]
            p2y = gbuf[7, sl]
            p2z = gbuf[8, sl]
            p3x = gbuf[9, sl]
            p3y = gbuf[10, sl]
            p3z = gbuf[11, sl]

            v1x = p0x - p1x
            v1y = p0y - p1y
            v1z = p0z - p1z
            v2x = p2x - p1x
            v2y = p2y - p1y
            v2z = p2z - p1z
            v3x = p2x - p3x
            v3y = p2y - p3y
            v3z = p2z - p3z

            c12x = v1y * v2z - v1z * v2y
            c12y = v1z * v2x - v1x * v2z
            c12z = v1x * v2y - v1y * v2x
            c23x = v2y * v3z - v2z * v3y
            c23y = v2z * v3x - v2x * v3z
            c23z = v2x * v3y - v2y * v3x

            a2 = c12x * c12x + c12y * c12y + c12z * c12z
            b2 = c23x * c23x + c23y * c23y + c23z * c23z
            dd = c12x * c23x + c12y * c23y + c12z * c23z
            tt = v1x * c23x + v1y * c23y + v1z * c23z

            q = jnp.maximum(a2 * b2, np.float32(1e-24))
            r = _rsqrt(q)
            c = jnp.clip(dd * r, np.float32(-1.0), np.float32(1.0))
            om = f1 - c * c
            sm = om * _rsqrt(jnp.maximum(om, np.float32(1e-30)))
            s = jnp.where(tt < np.float32(0.0), -sm, sm)

            fv = f_v[sl]
            pv = per_v[sl]
            phv = ph_v[sl]

            t = phv - _HALF_PI
            t2 = t * t
            sp = np.float32(_SIN[4])
            for cf in (_SIN[3], _SIN[2], _SIN[1], _SIN[0]):
                sp = sp * t2 + np.float32(cf)
            sp = sp * t
            cp = np.float32(_COS[5])
            for cf in (_COS[4], _COS[3], _COS[2], _COS[1], _COS[0]):
                cp = cp * t2 + np.float32(cf)
            cpsi = -sp
            spsi = cp

            cn, sn = c, s
            ck, sk = c, s
            for kk2 in range(2, 7):
                ck, sk = ck * c - sk * s, sk * c + ck * s
                sel = pv == np.float32(kk2)
                cn = jnp.where(sel, ck, cn)
                sn = jnp.where(sel, sk, sn)

            val = fv * (f1 + cn * cpsi + sn * spsi)
            return acc + val

        return lax.fori_loop(0, GROUPS, group_body, acc, unroll=False)

    acc = lax.fori_loop(0, NCHUNK, chunk_body, jnp.zeros((L,), jnp.float32),
                        unroll=False)
    acc_v[...] = acc
    pltpu.sync_copy(acc_v, out_ref.at[wid])


_sc_call = functools.partial(
    pl.kernel,
    out_type=jax.ShapeDtypeStruct((NW, L), jnp.float32),
    mesh=plsc.VectorSubcoreMesh(core_axis_name="c", subcore_axis_name="s",
                                num_cores=NC, num_subcores=NS),
    scratch_types=[
        pltpu.VMEM((NBLK_CHUNK, NSTR, BLK), jnp.int32),
        pltpu.VMEM((CHUNK,), jnp.float32),
        pltpu.VMEM((CHUNK,), jnp.float32),
        pltpu.VMEM((CHUNK,), jnp.float32),
        pltpu.VMEM((NSTR, CHUNK), jnp.float32),
        pltpu.VMEM((L,), jnp.float32),
        pltpu.SemaphoreType.DMA,
    ],
)(_sc_body)


@jax.jit
def kernel(coords, i, j, k, l, force, period, phase):
    cflat = coords.reshape(N_ATOMS * 3)
    pad = NPAD - N_DIH

    def pad_i32(x, val):
        return jnp.concatenate(
            [x.astype(jnp.int32), jnp.full((pad,), val, jnp.int32)])

    streams = []
    for idx in (i, j, k, l):
        base3 = pad_i32(idx, 0) * 3
        streams.extend([base3, base3 + 1, base3 + 2])
    idx_all = jnp.stack(streams)
    idx_all = idx_all.reshape(NSTR, NPAD // BLK, BLK).transpose(1, 0, 2)

    f2 = jnp.concatenate([force, jnp.zeros((pad,), jnp.float32)])
    p2 = jnp.concatenate([period, jnp.ones((pad,), jnp.float32)])
    ph2 = jnp.concatenate([phase, jnp.zeros((pad,), jnp.float32)])
    partials = _sc_call(cflat, idx_all, f2, p2, ph2)
    return jnp.sum(partials)
